# baseline (speedup 1.0000x reference)
import jax
import jax.numpy as jnp
from jax.experimental import pallas as pl
from jax.experimental.pallas import tpu as pltpu

_NE = 1024
_D = 64
_P = 1024
_B = 16
_BETA = 0.25


def _vq_body(ze_ref, emb_ref, st_ref, idx_ref, loss_ref):
    b = pl.program_id(0)
    ze = ze_ref[0].reshape(_D, _P)
    emb = emb_ref[...]
    es = jnp.dot(emb * emb, jnp.ones((_D, 1), jnp.float32),
                 preferred_element_type=jnp.float32)
    zs = jnp.sum(ze * ze, axis=0, keepdims=True)
    m = jnp.dot(emb, ze, preferred_element_type=jnp.float32)
    d = (zs + es) - 2.0 * m
    minv = jnp.min(d, axis=0, keepdims=True)
    iota = jax.lax.broadcasted_iota(
        jnp.int32, (_NE, _P), 0).astype(jnp.float32)
    cand = jnp.where(d == minv, iota, jnp.float32(2.0 * _NE))
    idxf = jnp.min(cand, axis=0, keepdims=True)
    idx_ref[0] = idxf.astype(jnp.int32)
    onehot = (cand == idxf).astype(jnp.float32)
    zq = jax.lax.dot_general(
        emb, onehot, dimension_numbers=(((0,), (0,)), ((), ())),
        preferred_element_type=jnp.float32)
    diff = zq - ze
    st_ref[0] = (ze + diff).reshape(_D, _P // 32, 32)
    part = jnp.sum(diff * diff).reshape(1, 1)

    @pl.when(b == 0)
    def _():
        loss_ref[...] = part

    @pl.when(b != 0)
    def _():
        loss_ref[...] = loss_ref[...] + part


def kernel(ze, embedding):
    B, L, H, W = ze.shape

    st, idx, loss_sum = pl.pallas_call(
        _vq_body,
        grid=(B,),
        in_specs=[
            pl.BlockSpec((1, L, H, W), lambda b: (b, 0, 0, 0)),
            pl.BlockSpec((_NE, _D), lambda b: (0, 0)),
        ],
        out_specs=[
            pl.BlockSpec((1, L, H, W), lambda b: (b, 0, 0, 0)),
            pl.BlockSpec((1, 1, _P), lambda b: (b, 0, 0)),
            pl.BlockSpec((1, 1), lambda b: (0, 0)),
        ],
        out_shape=[
            jax.ShapeDtypeStruct((B, L, H, W), jnp.float32),
            jax.ShapeDtypeStruct((B, 1, _P), jnp.int32),
            jax.ShapeDtypeStruct((1, 1), jnp.float32),
        ],
    )(ze, embedding)

    z_q_st = st
    n = float(B * L * H * W)
    mean_sq = loss_sum[0, 0] / n
    loss = mean_sq + _BETA * mean_sq
    min_idx = idx.reshape(-1, 1)
    return (z_q_st, loss, min_idx)


# speedup vs baseline: 1.4837x; 1.4837x over previous
import jax
import jax.numpy as jnp
from jax.experimental import pallas as pl
from jax.experimental.pallas import tpu as pltpu

_NE = 1024
_D = 64
_P = 1024
_B = 16
_BETA = 0.25


def _vq_body(ze_ref, embt_ref, st_ref, idx_ref, loss_ref):
    b = pl.program_id(0)
    ze = ze_ref[0]
    embt = embt_ref[...]
    es = jnp.sum(embt * embt, axis=0, keepdims=True)
    zs = jnp.sum(ze * ze, axis=1, keepdims=True)
    m = jnp.dot(ze, embt, preferred_element_type=jnp.float32)
    d = (zs + es) - 2.0 * m
    minv = jnp.min(d, axis=1, keepdims=True)
    iota = jax.lax.broadcasted_iota(
        jnp.int32, (_P, _NE), 1).astype(jnp.float32)
    cand = jnp.where(d == minv, iota, jnp.float32(2.0 * _NE))
    idxf = jnp.min(cand, axis=1, keepdims=True)
    onehot = (cand == idxf).astype(jnp.float32)
    iota_row = jax.lax.broadcasted_iota(
        jnp.int32, (1, _NE), 1).astype(jnp.float32)
    idx_row = jax.lax.dot_general(
        iota_row, onehot, dimension_numbers=(((1,), (1,)), ((), ())),
        preferred_element_type=jnp.float32)
    idx_ref[0] = idx_row.astype(jnp.int32)
    zq = jax.lax.dot_general(
        onehot, embt, dimension_numbers=(((1,), (1,)), ((), ())),
        preferred_element_type=jnp.float32)
    diff = zq - ze
    st_ref[0] = ze + diff
    part = jnp.sum(diff * diff).reshape(1, 1)

    @pl.when(b == 0)
    def _():
        loss_ref[...] = part

    @pl.when(b != 0)
    def _():
        loss_ref[...] = loss_ref[...] + part


def kernel(ze, embedding):
    B, L, H, W = ze.shape
    ze_r = jnp.transpose(ze, (0, 2, 3, 1)).reshape(B, H * W, L)
    embt = embedding.T

    st, idx, loss_sum = pl.pallas_call(
        _vq_body,
        grid=(B,),
        in_specs=[
            pl.BlockSpec((1, _P, _D), lambda b: (b, 0, 0)),
            pl.BlockSpec((_D, _NE), lambda b: (0, 0)),
        ],
        out_specs=[
            pl.BlockSpec((1, _P, _D), lambda b: (b, 0, 0)),
            pl.BlockSpec((1, 1, _P), lambda b: (b, 0, 0)),
            pl.BlockSpec((1, 1), lambda b: (0, 0)),
        ],
        out_shape=[
            jax.ShapeDtypeStruct((B, _P, _D), jnp.float32),
            jax.ShapeDtypeStruct((B, 1, _P), jnp.int32),
            jax.ShapeDtypeStruct((1, 1), jnp.float32),
        ],
    )(ze_r, embt)

    z_q_st = jnp.transpose(st.reshape(B, H, W, L), (0, 3, 1, 2))
    n = float(B * L * H * W)
    mean_sq = loss_sum[0, 0] / n
    loss = mean_sq + _BETA * mean_sq
    min_idx = idx.reshape(-1, 1)
    return (z_q_st, loss, min_idx)
